# baseline (device time: 30456 ns/iter reference)
import jax
import jax.numpy as jnp
from jax import lax
from jax.experimental import pallas as pl
from jax.experimental.pallas import tpu as pltpu


def kernel(ids, E):
    v_local, d = E.shape
    (t_total,) = ids.shape

    def body(ids_ref, e_ref, out_ref, tok_ref, row_ref,
             copy_sem, send_sem, recv_sem):
        x = lax.axis_index("x")
        y = lax.axis_index("y")
        z = lax.axis_index("z")
        partner = (1 - x, y, z)
        lo = x * v_local

        barrier_sem = pltpu.get_barrier_semaphore()
        pl.semaphore_signal(
            barrier_sem, inc=1,
            device_id=partner, device_id_type=pl.DeviceIdType.MESH,
        )
        pl.semaphore_wait(barrier_sem, 1)

        blk = 128
        n_blk = t_total // blk

        def comp(t, c):
            r = ids_ref[t] - lo
            mine = r.astype(jnp.uint32) < jnp.uint32(v_local)
            tok_ref[c] = t
            row_ref[c] = r
            return c + mine.astype(jnp.int32)

        def issue(i, c):
            src = e_ref.at[pl.ds(row_ref[i], 1), :]
            dst = out_ref.at[pl.ds(tok_ref[i], 1), :]
            pltpu.make_async_remote_copy(
                src_ref=src,
                dst_ref=dst,
                send_sem=send_sem,
                recv_sem=recv_sem,
                device_id=partner,
                device_id_type=pl.DeviceIdType.MESH,
            ).start()
            pltpu.make_async_copy(src, dst, copy_sem).start()
            return c

        def block(b, c0):
            c1 = lax.fori_loop(b * blk, (b + 1) * blk, comp, c0)
            lax.fori_loop(c0, c1, issue, jnp.int32(0))
            return c1

        k = lax.fori_loop(0, n_blk, block, jnp.int32(0))

        def drain_mine(w):
            def f(_, c):
                pltpu.make_async_copy(
                    e_ref.at[pl.ds(0, w), :], out_ref.at[pl.ds(0, w), :],
                    copy_sem,
                ).wait()
                pltpu.make_async_remote_copy(
                    src_ref=e_ref.at[pl.ds(0, w), :],
                    dst_ref=out_ref.at[pl.ds(0, w), :],
                    send_sem=send_sem,
                    recv_sem=recv_sem,
                    device_id=partner,
                    device_id_type=pl.DeviceIdType.MESH,
                ).wait_send()
                return c
            return f

        def drain_recv(w):
            def f(_, c):
                pltpu.make_async_remote_copy(
                    src_ref=e_ref.at[pl.ds(0, w), :],
                    dst_ref=out_ref.at[pl.ds(0, w), :],
                    send_sem=send_sem,
                    recv_sem=recv_sem,
                    device_id=partner,
                    device_id_type=pl.DeviceIdType.MESH,
                ).wait_recv()
                return c
            return f

        n_recv = t_total - k
        lax.fori_loop(0, k // 8, drain_mine(8), 0)
        lax.fori_loop(0, k % 8, drain_mine(1), 0)
        lax.fori_loop(0, n_recv // 8, drain_recv(8), 0)
        lax.fori_loop(0, n_recv % 8, drain_recv(1), 0)

    return pl.pallas_call(
        body,
        out_shape=jax.ShapeDtypeStruct((t_total, d), jnp.float32),
        in_specs=[
            pl.BlockSpec(memory_space=pltpu.SMEM),
            pl.BlockSpec(memory_space=pl.ANY),
        ],
        out_specs=pl.BlockSpec(memory_space=pl.ANY),
        scratch_shapes=[
            pltpu.SMEM((t_total,), jnp.int32),
            pltpu.SMEM((t_total,), jnp.int32),
            pltpu.SemaphoreType.DMA,
            pltpu.SemaphoreType.DMA,
            pltpu.SemaphoreType.DMA,
        ],
        compiler_params=pltpu.CompilerParams(collective_id=0),
    )(ids, E)


# device time: 30075 ns/iter; 1.0127x vs baseline; 1.0127x over previous
import jax
import jax.numpy as jnp
from jax import lax
from jax.experimental import pallas as pl
from jax.experimental.pallas import tpu as pltpu


def kernel(ids, E):
    v_local, d = E.shape
    (t_total,) = ids.shape

    def body(ids_ref, e_ref, out_ref, tok_ref, row_ref,
             copy_sem, send_sem, recv_sem):
        x = lax.axis_index("x")
        y = lax.axis_index("y")
        z = lax.axis_index("z")
        partner = (1 - x, y, z)
        lo = x * v_local

        barrier_sem = pltpu.get_barrier_semaphore()
        pl.semaphore_signal(
            barrier_sem, inc=1,
            device_id=partner, device_id_type=pl.DeviceIdType.MESH,
        )
        pl.semaphore_wait(barrier_sem, 1)

        blk = 128
        n_blk = t_total // blk

        def comp(j, c):
            for u in range(4):
                t = j * 4 + u
                r = ids_ref[t] - lo
                mine = r.astype(jnp.uint32) < jnp.uint32(v_local)
                tok_ref[c] = t
                row_ref[c] = r
                c = c + mine.astype(jnp.int32)
            return c

        def issue(i, c):
            src = e_ref.at[pl.ds(row_ref[i], 1), :]
            dst = out_ref.at[pl.ds(tok_ref[i], 1), :]
            pltpu.make_async_remote_copy(
                src_ref=src,
                dst_ref=dst,
                send_sem=send_sem,
                recv_sem=recv_sem,
                device_id=partner,
                device_id_type=pl.DeviceIdType.MESH,
            ).start()
            pltpu.make_async_copy(src, dst, copy_sem).start()
            return c

        def block(b, c0):
            c1 = lax.fori_loop(b * (blk // 4), (b + 1) * (blk // 4), comp, c0)
            lax.fori_loop(c0, c1, issue, jnp.int32(0))
            return c1

        k = lax.fori_loop(0, n_blk, block, jnp.int32(0))

        def drain_mine(w):
            def f(_, c):
                pltpu.make_async_copy(
                    e_ref.at[pl.ds(0, w), :], out_ref.at[pl.ds(0, w), :],
                    copy_sem,
                ).wait()
                pltpu.make_async_remote_copy(
                    src_ref=e_ref.at[pl.ds(0, w), :],
                    dst_ref=out_ref.at[pl.ds(0, w), :],
                    send_sem=send_sem,
                    recv_sem=recv_sem,
                    device_id=partner,
                    device_id_type=pl.DeviceIdType.MESH,
                ).wait_send()
                return c
            return f

        def drain_recv(w):
            def f(_, c):
                pltpu.make_async_remote_copy(
                    src_ref=e_ref.at[pl.ds(0, w), :],
                    dst_ref=out_ref.at[pl.ds(0, w), :],
                    send_sem=send_sem,
                    recv_sem=recv_sem,
                    device_id=partner,
                    device_id_type=pl.DeviceIdType.MESH,
                ).wait_recv()
                return c
            return f

        n_recv = t_total - k
        lax.fori_loop(0, k // 8, drain_mine(8), 0)
        lax.fori_loop(0, k % 8, drain_mine(1), 0)
        lax.fori_loop(0, n_recv // 8, drain_recv(8), 0)
        lax.fori_loop(0, n_recv % 8, drain_recv(1), 0)

    return pl.pallas_call(
        body,
        out_shape=jax.ShapeDtypeStruct((t_total, d), jnp.float32),
        in_specs=[
            pl.BlockSpec(memory_space=pltpu.SMEM),
            pl.BlockSpec(memory_space=pl.ANY),
        ],
        out_specs=pl.BlockSpec(memory_space=pl.ANY),
        scratch_shapes=[
            pltpu.SMEM((t_total,), jnp.int32),
            pltpu.SMEM((t_total,), jnp.int32),
            pltpu.SemaphoreType.DMA,
            pltpu.SemaphoreType.DMA,
            pltpu.SemaphoreType.DMA,
        ],
        compiler_params=pltpu.CompilerParams(collective_id=0),
    )(ids, E)
